# baseline (speedup 1.0000x reference)
import functools

import jax
import jax.numpy as jnp
from jax import lax
from jax.experimental import pallas as pl
from jax.experimental.pallas import tpu as pltpu
from jax.experimental.pallas import tpu_sc as plsc

NC = 2
NS = 16
NW = NC * NS

B = 16384
F = 26
DIM = 32
BLK = B // NW
QI = 128
NQ = BLK // QI
LANES = 16


def _stage_chunk(xT_hbm, off_v, table_hbm, cidx_v, rows_v, gsem, f, buf, b0):
    pltpu.sync_copy(xT_hbm.at[f, pl.ds(b0, BLK)], cidx_v.at[buf])
    off_row = off_v[f, :]

    def add_one(t, carry):
        sl = pl.ds(t * LANES, LANES)
        cidx_v[buf, sl] = cidx_v[buf, sl] + off_row
        return carry

    lax.fori_loop(0, BLK // LANES, add_one, 0)
    for q in range(NQ):
        pltpu.async_copy(
            table_hbm.at[cidx_v.at[buf, pl.ds(q * QI, QI)]],
            rows_v.at[buf, pl.ds(q * QI, QI)],
            gsem,
        )


def _body(xT_hbm, off_hbm, table_hbm, out_hbm, cidx_v, off_v, rows_v, tbuf_v,
          gsem, ssem):
    wid = lax.axis_index("s") * NC + lax.axis_index("c")
    b0 = wid * BLK

    pltpu.sync_copy(off_hbm, off_v)
    _stage_chunk(xT_hbm, off_v, table_hbm, cidx_v, rows_v, gsem, 0, 0, b0)

    iota = lax.iota(jnp.int32, LANES)
    dcols = [jnp.full((LANES,), d, jnp.int32) for d in range(DIM)]

    def chunk_step(k, carry):
        cb = k & 1

        @pl.when(k + 1 < F)
        def _stage_next():
            _stage_chunk(
                xT_hbm, off_v, table_hbm, cidx_v, rows_v, gsem, k + 1, 1 - cb, b0
            )

        for q in range(NQ):
            pltpu.make_async_copy(
                table_hbm.at[cidx_v.at[cb, pl.ds(q * QI, QI)]],
                rows_v.at[cb, pl.ds(q * QI, QI)],
                gsem,
            ).wait()

        @pl.when(k >= 2)
        def _drain_store():
            pltpu.make_async_copy(
                tbuf_v.at[cb], out_hbm.at[k, :, pl.ds(b0, BLK)], ssem
            ).wait()

        def tr_one(t, carry):
            row_idx = t * LANES + iota
            for d in range(DIM):
                v = plsc.load_gather(rows_v.at[cb], [row_idx, dcols[d]])
                tbuf_v[cb, d, pl.ds(t * LANES, LANES)] = v
            return carry

        lax.fori_loop(0, BLK // LANES, tr_one, 0)

        pltpu.async_copy(tbuf_v.at[cb], out_hbm.at[k, :, pl.ds(b0, BLK)], ssem)
        return carry

    lax.fori_loop(0, F, chunk_step, 0)

    for _ in range(2):
        pltpu.make_async_copy(
            tbuf_v.at[0], out_hbm.at[0, :, pl.ds(b0, BLK)], ssem
        ).wait()


@jax.jit
def _run(xT, off_bcast, table):
    mesh = plsc.VectorSubcoreMesh(
        core_axis_name="c", subcore_axis_name="s", num_cores=NC, num_subcores=NS
    )
    fn = pl.kernel(
        _body,
        out_type=jax.ShapeDtypeStruct((F, DIM, B), jnp.float32),
        mesh=mesh,
        scratch_types=[
            pltpu.VMEM((2, BLK), jnp.int32),
            pltpu.VMEM((F, LANES), jnp.int32),
            pltpu.VMEM((2, BLK, DIM), jnp.float32),
            pltpu.VMEM((2, DIM, BLK), jnp.float32),
            pltpu.SemaphoreType.DMA,
            pltpu.SemaphoreType.DMA,
        ],
        compiler_params=pltpu.CompilerParams(
            use_tc_tiling_on_sc=False, needs_layout_passes=False
        ),
    )
    return fn(xT, off_bcast, table)


def kernel(x_categ, table, offsets):
    xT = x_categ.astype(jnp.int32).T
    off_bcast = jnp.broadcast_to(
        offsets.astype(jnp.int32)[:, None], (F, LANES)
    )
    outT = _run(xT, off_bcast, table)
    return outT.transpose(2, 0, 1)


# speedup vs baseline: 1.2267x; 1.2267x over previous
import functools

import jax
import jax.numpy as jnp
from jax import lax
from jax.experimental import pallas as pl
from jax.experimental.pallas import tpu as pltpu
from jax.experimental.pallas import tpu_sc as plsc

NC = 2
NS = 16
NW = NC * NS

B = 16384
F = 26
DIM = 32
BLK = B // NW
QI = 128
NQ = BLK // QI
LANES = 16
NBUF = 3


def _fire_gathers(table_hbm, cidx_v, rows_v, gsem, f, buf):
    for q in range(NQ):
        pltpu.async_copy(
            table_hbm.at[cidx_v.at[f, pl.ds(q * QI, QI)]],
            rows_v.at[buf, pl.ds(q * QI, QI)],
            gsem,
        )


def _wait_gathers(table_hbm, cidx_v, rows_v, gsem, f, buf):
    for q in range(NQ):
        pltpu.make_async_copy(
            table_hbm.at[cidx_v.at[f, pl.ds(q * QI, QI)]],
            rows_v.at[buf, pl.ds(q * QI, QI)],
            gsem,
        ).wait()


def _body(xT_hbm, off_hbm, table_hbm, out_hbm, cidx_v, off_v, rows_v, gsem, ssem):
    wid = lax.axis_index("s") * NC + lax.axis_index("c")
    b0 = wid * BLK

    pltpu.sync_copy(xT_hbm.at[:, pl.ds(b0, BLK)], cidx_v)
    pltpu.sync_copy(off_hbm, off_v)

    def add_f(f, carry):
        off_row = off_v[f, :]

        def add_t(t, c2):
            sl = pl.ds(t * LANES, LANES)
            cidx_v[f, sl] = cidx_v[f, sl] + off_row
            return c2

        return lax.fori_loop(0, BLK // LANES, add_t, carry)

    lax.fori_loop(0, F, add_f, 0)

    for f in range(2):
        _fire_gathers(table_hbm, cidx_v, rows_v, gsem, f, f)

    def chunk_step(j, carry):
        cb = lax.rem(j, NBUF)

        @pl.when(j + 2 < F)
        def _stage_ahead():
            nb = lax.rem(j + 2, NBUF)

            @pl.when(j >= 1)
            def _drain_store():
                pltpu.make_async_copy(
                    rows_v.at[0], out_hbm.at[0, pl.ds(b0, BLK)], ssem
                ).wait()

            _fire_gathers(table_hbm, cidx_v, rows_v, gsem, j + 2, nb)

        _wait_gathers(table_hbm, cidx_v, rows_v, gsem, j, cb)
        pltpu.async_copy(rows_v.at[cb], out_hbm.at[j, pl.ds(b0, BLK)], ssem)
        return carry

    lax.fori_loop(0, F, chunk_step, 0)

    for _ in range(NBUF):
        pltpu.make_async_copy(
            rows_v.at[0], out_hbm.at[0, pl.ds(b0, BLK)], ssem
        ).wait()


@jax.jit
def _run(xT, off_bcast, table):
    mesh = plsc.VectorSubcoreMesh(
        core_axis_name="c", subcore_axis_name="s", num_cores=NC, num_subcores=NS
    )
    fn = pl.kernel(
        _body,
        out_type=jax.ShapeDtypeStruct((F, B, DIM), jnp.float32),
        mesh=mesh,
        scratch_types=[
            pltpu.VMEM((F, BLK), jnp.int32),
            pltpu.VMEM((F, LANES), jnp.int32),
            pltpu.VMEM((NBUF, BLK, DIM), jnp.float32),
            pltpu.SemaphoreType.DMA,
            pltpu.SemaphoreType.DMA,
        ],
        compiler_params=pltpu.CompilerParams(use_tc_tiling_on_sc=False),
    )
    return fn(xT, off_bcast, table)


def kernel(x_categ, table, offsets):
    xT = x_categ.astype(jnp.int32).T
    off_bcast = jnp.broadcast_to(
        offsets.astype(jnp.int32)[:, None], (F, LANES)
    )
    out2 = _run(xT, off_bcast, table)
    return out2.transpose(1, 0, 2)
